# baseline (speedup 1.0000x reference)
import functools

import jax
import jax.numpy as jnp
from jax import lax
from jax.experimental import pallas as pl
from jax.experimental.pallas import tpu as pltpu
from jax.experimental.pallas import tpu_sc as plsc

B = 16
L = 2048
P = 4194304
ALPHA = 1e-4
BETA = 1.0
MAX_STD = 1000.0
MIN_STD = 1.0

NC = 2
NS = 16
LANES = 16
NW = NC * NS
PER_W = P // NW
CHUNK = 32768
NCH = PER_W // CHUNK
NACC = 8


@functools.cache
def _l1_partials_kernel():
    return pl.kernel(
        _l1_body,
        mesh=plsc.VectorSubcoreMesh(core_axis_name="c", subcore_axis_name="s"),
        out_type=jax.ShapeDtypeStruct((NW, LANES), jnp.float32),
        scratch_types=[
            pltpu.VMEM((CHUNK,), jnp.float32),
            pltpu.VMEM((CHUNK,), jnp.float32),
            pltpu.VMEM((LANES,), jnp.float32),
            pltpu.SemaphoreType.DMA,
            pltpu.SemaphoreType.DMA,
        ],
    )


def _l1_body(params_hbm, out_hbm, buf_a, buf_b, outv, sem_a, sem_b):
    c = lax.axis_index("c")
    s = lax.axis_index("s")
    wid = s * NC + c
    base = wid * PER_W
    bufs = (buf_a, buf_b)
    sems = (sem_a, sem_b)

    copies = [None, None]
    copies[0] = pltpu.async_copy(
        params_hbm.at[pl.ds(base, CHUNK)], bufs[0], sems[0])

    accs = tuple(jnp.zeros((LANES,), jnp.float32) for _ in range(NACC))
    span = LANES * NACC
    for ch in range(NCH):
        cur = ch % 2
        if ch + 1 < NCH:
            nxt = (ch + 1) % 2
            copies[nxt] = pltpu.async_copy(
                params_hbm.at[pl.ds(base + (ch + 1) * CHUNK, CHUNK)],
                bufs[nxt], sems[nxt])
        copies[cur].wait()
        buf = bufs[cur]

        def body(j, accs):
            s0 = j * span
            return tuple(
                a + jnp.abs(buf[pl.ds(s0 + k * LANES, LANES)])
                for k, a in enumerate(accs))

        accs = lax.fori_loop(0, CHUNK // span, body, accs)

    total = accs[0]
    for a in accs[1:]:
        total = total + a
    outv[...] = total
    pltpu.sync_copy(outv, out_hbm.at[wid])


def _tc_body(logits_ref, labels_ref, aw_ref, nll_ref, ap_ref):
    logits = logits_ref[...]
    labels = labels_ref[...]
    aw = aw_ref[...]

    m = jnp.max(logits, axis=1, keepdims=True)
    z = logits - m
    lse = jnp.log(jnp.sum(jnp.exp(z), axis=1, keepdims=True))
    logp = z - lse
    sel = jnp.where(labels == 1, logp[:, 1:2], logp[:, 0:1])
    nll = -jnp.mean(sel)

    xi = lax.broadcasted_iota(jnp.int32, (B, L), 1)
    x = (xi.astype(jnp.float32) + 1.0) * (1.0 / L)
    sums = jnp.sum(aw, axis=1, keepdims=True)
    means = jnp.sum(x * aw, axis=1, keepdims=True) / sums
    std = jnp.where(labels.astype(jnp.float32) == 1.0, MIN_STD, MAX_STD) * (1.0 / L)
    t = (x - means) / std
    r_hat = jnp.exp(-0.5 * t * t) / (std * jnp.sqrt(2.0 * jnp.pi))
    rs = r_hat / (jnp.sum(r_hat, axis=1, keepdims=True) + 1e-6)
    diff = aw - rs
    ap = (BETA / 2.0) * jnp.mean(diff * diff)

    nll_ref[...] = nll.reshape(1, 1)
    ap_ref[...] = ap.reshape(1, 1)


def _combine_body(part_ref, nll_ref, ap_ref, loss_ref, nll_out_ref):
    nll = nll_ref[0, 0]
    l1 = jnp.sum(part_ref[...])
    loss = nll + (ALPHA / 2.0) * l1 + ap_ref[0, 0]
    loss_ref[...] = loss.reshape(1, 1)
    nll_out_ref[...] = nll.reshape(1, 1)


_combine_call = pl.pallas_call(
    _combine_body,
    out_shape=(
        jax.ShapeDtypeStruct((1, 1), jnp.float32),
        jax.ShapeDtypeStruct((1, 1), jnp.float32),
    ),
)


_tc_call = pl.pallas_call(
    _tc_body,
    out_shape=(
        jax.ShapeDtypeStruct((1, 1), jnp.float32),
        jax.ShapeDtypeStruct((1, 1), jnp.float32),
    ),
)


def kernel(logits, labels, params, lengths, attn_weights):
    del lengths
    partials = _l1_partials_kernel()(params)
    nll, ap = _tc_call(
        logits, labels.reshape(B, 1), attn_weights.reshape(B, L))
    loss, nll_out = _combine_call(partials, nll, ap)
    return (loss.reshape(()), nll_out.reshape(()))


# speedup vs baseline: 1.0203x; 1.0203x over previous
import functools

import jax
import jax.numpy as jnp
from jax import lax
from jax.experimental import pallas as pl
from jax.experimental.pallas import tpu as pltpu
from jax.experimental.pallas import tpu_sc as plsc

B = 16
L = 2048
P = 4194304
ALPHA = 1e-4
BETA = 1.0
MAX_STD = 1000.0
MIN_STD = 1.0

NC = 2
NS = 16
LANES = 16
NW = NC * NS
PER_W = P // NW
CHUNK = 32768
NCH = PER_W // CHUNK
NACC = 8


@functools.cache
def _l1_partials_kernel():
    return pl.kernel(
        _l1_body,
        mesh=plsc.VectorSubcoreMesh(core_axis_name="c", subcore_axis_name="s"),
        out_type=jax.ShapeDtypeStruct((NW, LANES), jnp.float32),
        scratch_types=[
            pltpu.VMEM((CHUNK,), jnp.float32),
            pltpu.VMEM((CHUNK,), jnp.float32),
            pltpu.VMEM((CHUNK,), jnp.float32),
            pltpu.VMEM((LANES,), jnp.float32),
            pltpu.SemaphoreType.DMA,
            pltpu.SemaphoreType.DMA,
            pltpu.SemaphoreType.DMA,
        ],
    )


def _l1_body(params_hbm, out_hbm, buf_a, buf_b, buf_c, outv,
             sem_a, sem_b, sem_c):
    c = lax.axis_index("c")
    s = lax.axis_index("s")
    wid = s * NC + c
    base = wid * PER_W
    bufs = (buf_a, buf_b, buf_c)
    sems = (sem_a, sem_b, sem_c)
    nbuf = 3

    copies = [None] * nbuf
    for k in range(min(2, NCH)):
        copies[k] = pltpu.async_copy(
            params_hbm.at[pl.ds(base + k * CHUNK, CHUNK)], bufs[k], sems[k])

    accs = tuple(jnp.zeros((LANES,), jnp.float32) for _ in range(NACC))
    span = LANES * NACC
    for ch in range(NCH):
        cur = ch % nbuf
        if ch + 2 < NCH:
            nxt = (ch + 2) % nbuf
            copies[nxt] = pltpu.async_copy(
                params_hbm.at[pl.ds(base + (ch + 2) * CHUNK, CHUNK)],
                bufs[nxt], sems[nxt])
        copies[cur].wait()
        buf = bufs[cur]

        def body(j, accs):
            s0 = j * span
            return tuple(
                a + jnp.abs(buf[pl.ds(s0 + k * LANES, LANES)])
                for k, a in enumerate(accs))

        accs = lax.fori_loop(0, CHUNK // span, body, accs)

    total = accs[0]
    for a in accs[1:]:
        total = total + a
    outv[...] = total
    pltpu.sync_copy(outv, out_hbm.at[wid])


def _tc_body(logits_ref, labels_ref, aw_ref, part_ref, loss_ref, nll_ref):
    logits = logits_ref[...]
    labels = labels_ref[...]
    aw = aw_ref[...]
    parts = part_ref[...]

    m = jnp.max(logits, axis=1, keepdims=True)
    z = logits - m
    lse = jnp.log(jnp.sum(jnp.exp(z), axis=1, keepdims=True))
    logp = z - lse
    sel = jnp.where(labels == 1, logp[:, 1:2], logp[:, 0:1])
    nll = -jnp.mean(sel)

    xi = lax.broadcasted_iota(jnp.int32, (B, L), 1)
    x = (xi.astype(jnp.float32) + 1.0) * (1.0 / L)
    sums = jnp.sum(aw, axis=1, keepdims=True)
    means = jnp.sum(x * aw, axis=1, keepdims=True) / sums
    std = jnp.where(labels.astype(jnp.float32) == 1.0, MIN_STD, MAX_STD) * (1.0 / L)
    t = (x - means) / std
    r_hat = jnp.exp(-0.5 * t * t) / (std * jnp.sqrt(2.0 * jnp.pi))
    rs = r_hat / (jnp.sum(r_hat, axis=1, keepdims=True) + 1e-6)
    diff = aw - rs
    ap = (BETA / 2.0) * jnp.mean(diff * diff)

    l1 = jnp.sum(parts)
    loss = nll + (ALPHA / 2.0) * l1 + ap
    loss_ref[...] = loss.reshape(1, 1)
    nll_ref[...] = nll.reshape(1, 1)


_tc_call = pl.pallas_call(
    _tc_body,
    out_shape=(
        jax.ShapeDtypeStruct((1, 1), jnp.float32),
        jax.ShapeDtypeStruct((1, 1), jnp.float32),
    ),
)


def kernel(logits, labels, params, lengths, attn_weights):
    del lengths
    partials = _l1_partials_kernel()(params)
    loss, nll = _tc_call(
        logits, labels.reshape(B, 1), attn_weights.reshape(B, L), partials)
    return (loss.reshape(()), nll.reshape(()))


# speedup vs baseline: 1.0205x; 1.0002x over previous
import functools

import jax
import jax.numpy as jnp
from jax import lax
from jax.experimental import pallas as pl
from jax.experimental.pallas import tpu as pltpu
from jax.experimental.pallas import tpu_sc as plsc

B = 16
L = 2048
P = 4194304
ALPHA = 1e-4
BETA = 1.0
MAX_STD = 1000.0
MIN_STD = 1.0

NC = 2
NS = 16
LANES = 16
NW = NC * NS
PER_W = P // NW
CHUNK = 32768
NCH = PER_W // CHUNK
NACC = 8


@functools.cache
def _l1_partials_kernel():
    return pl.kernel(
        _l1_body,
        mesh=plsc.VectorSubcoreMesh(core_axis_name="c", subcore_axis_name="s"),
        out_type=jax.ShapeDtypeStruct((NW, LANES), jnp.float32),
        scratch_types=[
            pltpu.VMEM((CHUNK,), jnp.float32),
            pltpu.VMEM((CHUNK,), jnp.float32),
            pltpu.VMEM((LANES,), jnp.float32),
            pltpu.SemaphoreType.DMA,
            pltpu.SemaphoreType.DMA,
        ],
    )


def _l1_body(params_hbm, out_hbm, buf_a, buf_b, outv, sem_a, sem_b):
    c = lax.axis_index("c")
    s = lax.axis_index("s")
    wid = s * NC + c
    base = wid * PER_W
    bufs = (buf_a, buf_b)
    sems = (sem_a, sem_b)

    copies = [None, None]
    copies[0] = pltpu.async_copy(
        params_hbm.at[pl.ds(base, CHUNK)], bufs[0], sems[0])

    accs = tuple(jnp.zeros((LANES,), jnp.float32) for _ in range(NACC))
    span = LANES * NACC
    for ch in range(NCH):
        cur = ch % 2
        if ch + 1 < NCH:
            nxt = (ch + 1) % 2
            copies[nxt] = pltpu.async_copy(
                params_hbm.at[pl.ds(base + (ch + 1) * CHUNK, CHUNK)],
                bufs[nxt], sems[nxt])
        copies[cur].wait()
        buf = bufs[cur]

        def body(j, accs):
            s0 = j * span
            return tuple(
                a + jnp.abs(buf[pl.ds(s0 + k * LANES, LANES)])
                for k, a in enumerate(accs))

        accs = lax.fori_loop(0, CHUNK // span, body, accs)

    total = accs[0]
    for a in accs[1:]:
        total = total + a
    outv[...] = total
    pltpu.sync_copy(outv, out_hbm.at[wid])


def _tc_body(logits_ref, labels_ref, aw_ref, part_ref, loss_ref, nll_ref):
    logits = logits_ref[...]
    labels = labels_ref[...]
    aw = aw_ref[...]
    parts = part_ref[...]

    m = jnp.max(logits, axis=1, keepdims=True)
    z = logits - m
    lse = jnp.log(jnp.sum(jnp.exp(z), axis=1, keepdims=True))
    logp = z - lse
    sel = jnp.where(labels == 1, logp[:, 1:2], logp[:, 0:1])
    nll = -jnp.mean(sel)

    xi = lax.broadcasted_iota(jnp.int32, (B, L), 1)
    x = (xi.astype(jnp.float32) + 1.0) * (1.0 / L)
    sums = jnp.sum(aw, axis=1, keepdims=True)
    means = jnp.sum(x * aw, axis=1, keepdims=True) / sums
    std = jnp.where(labels.astype(jnp.float32) == 1.0, MIN_STD, MAX_STD) * (1.0 / L)
    t = (x - means) / std
    r_hat = jnp.exp(-0.5 * t * t) / (std * jnp.sqrt(2.0 * jnp.pi))
    rs = r_hat / (jnp.sum(r_hat, axis=1, keepdims=True) + 1e-6)
    diff = aw - rs
    ap = (BETA / 2.0) * jnp.mean(diff * diff)

    l1 = jnp.sum(parts)
    loss = nll + (ALPHA / 2.0) * l1 + ap
    loss_ref[...] = loss.reshape(1, 1)
    nll_ref[...] = nll.reshape(1, 1)


_tc_call = pl.pallas_call(
    _tc_body,
    out_shape=(
        jax.ShapeDtypeStruct((1, 1), jnp.float32),
        jax.ShapeDtypeStruct((1, 1), jnp.float32),
    ),
)


def kernel(logits, labels, params, lengths, attn_weights):
    del lengths
    partials = _l1_partials_kernel()(params)
    loss, nll = _tc_call(
        logits, labels.reshape(B, 1), attn_weights.reshape(B, L), partials)
    return (loss.reshape(()), nll.reshape(()))


# speedup vs baseline: 1.0350x; 1.0143x over previous
import functools

import jax
import jax.numpy as jnp
from jax import lax
from jax.experimental import pallas as pl
from jax.experimental.pallas import tpu as pltpu
from jax.experimental.pallas import tpu_sc as plsc

B = 16
L = 2048
P = 4194304
ALPHA = 1e-4
BETA = 1.0
MAX_STD = 1000.0
MIN_STD = 1.0

NC = 2
NS = 16
LANES = 16
NW = NC * NS

P_SC = 2097152
PER_W = P_SC // NW
CHUNK = 16384
NCH = PER_W // CHUNK
NACC = 8

TC_BASE = P_SC
TC_ROW = 65536
TC_ROWS = 8
TCK = TC_ROW * TC_ROWS
NCH_TC = (P - P_SC) // TCK


@functools.cache
def _l1_partials_kernel():
    return pl.kernel(
        _l1_body,
        mesh=plsc.VectorSubcoreMesh(core_axis_name="c", subcore_axis_name="s"),
        out_type=jax.ShapeDtypeStruct((NW, LANES), jnp.float32),
        scratch_types=[
            pltpu.VMEM((CHUNK,), jnp.float32),
            pltpu.VMEM((CHUNK,), jnp.float32),
            pltpu.VMEM((LANES,), jnp.float32),
            pltpu.SemaphoreType.DMA,
            pltpu.SemaphoreType.DMA,
        ],
    )


def _l1_body(params_hbm, out_hbm, buf_a, buf_b, outv, sem_a, sem_b):
    c = lax.axis_index("c")
    s = lax.axis_index("s")
    wid = s * NC + c
    base = wid * PER_W
    bufs = (buf_a, buf_b)
    sems = (sem_a, sem_b)

    copies = [None, None]
    copies[0] = pltpu.async_copy(
        params_hbm.at[pl.ds(base, CHUNK)], bufs[0], sems[0])

    accs = tuple(jnp.zeros((LANES,), jnp.float32) for _ in range(NACC))
    span = LANES * NACC
    for ch in range(NCH):
        cur = ch % 2
        if ch + 1 < NCH:
            nxt = (ch + 1) % 2
            copies[nxt] = pltpu.async_copy(
                params_hbm.at[pl.ds(base + (ch + 1) * CHUNK, CHUNK)],
                bufs[nxt], sems[nxt])
        copies[cur].wait()
        buf = bufs[cur]

        def body(j, accs):
            s0 = j * span
            return tuple(
                a + jnp.abs(buf[pl.ds(s0 + k * LANES, LANES)])
                for k, a in enumerate(accs))

        accs = lax.fori_loop(0, CHUNK // span, body, accs)

    total = accs[0]
    for a in accs[1:]:
        total = total + a
    outv[...] = total
    pltpu.sync_copy(outv, out_hbm.at[wid])


def _tc1_body(logits_ref, labels_ref, aw_ref, params_ref,
              nll_ref, ap_ref, l1tc_ref, buf_a, buf_b, sem_a, sem_b):
    bufs = (buf_a, buf_b)
    sems = (sem_a, sem_b)

    def make_copies(ch, b):
        base = TC_BASE + ch * TCK
        return [pltpu.make_async_copy(
            params_ref.at[pl.ds(base + r * TC_ROW, TC_ROW)],
            bufs[b].at[r], sems[b]) for r in range(TC_ROWS)]

    pend = [None, None]
    for k in range(min(2, NCH_TC)):
        pend[k] = make_copies(k, k)
        for cp in pend[k]:
            cp.start()

    logits = logits_ref[...]
    labels = labels_ref[...]
    aw = aw_ref[...]

    m = jnp.max(logits, axis=1, keepdims=True)
    z = logits - m
    lse = jnp.log(jnp.sum(jnp.exp(z), axis=1, keepdims=True))
    logp = z - lse
    sel = jnp.where(labels == 1, logp[:, 1:2], logp[:, 0:1])
    nll = -jnp.mean(sel)

    xi = lax.broadcasted_iota(jnp.int32, (B, L), 1)
    x = (xi.astype(jnp.float32) + 1.0) * (1.0 / L)
    sums = jnp.sum(aw, axis=1, keepdims=True)
    means = jnp.sum(x * aw, axis=1, keepdims=True) / sums
    std = jnp.where(labels.astype(jnp.float32) == 1.0,
                    MIN_STD, MAX_STD) * (1.0 / L)
    t = (x - means) / std
    r_hat = jnp.exp(-0.5 * t * t) / (std * jnp.sqrt(2.0 * jnp.pi))
    rs = r_hat / (jnp.sum(r_hat, axis=1, keepdims=True) + 1e-6)
    diff = aw - rs
    ap = (BETA / 2.0) * jnp.mean(diff * diff)

    nll_ref[...] = nll.reshape(1, 1)
    ap_ref[...] = ap.reshape(1, 1)

    l1 = jnp.float32(0.0)
    for ch in range(NCH_TC):
        b = ch % 2
        for cp in pend[b]:
            cp.wait()
        l1 = l1 + jnp.sum(jnp.abs(bufs[b][...]))
        if ch + 2 < NCH_TC:
            pend[b] = make_copies(ch + 2, b)
            for cp in pend[b]:
                cp.start()
    l1tc_ref[...] = l1.reshape(1, 1)


_tc1_call = pl.pallas_call(
    _tc1_body,
    in_specs=[
        pl.BlockSpec(memory_space=pltpu.VMEM),
        pl.BlockSpec(memory_space=pltpu.VMEM),
        pl.BlockSpec(memory_space=pltpu.VMEM),
        pl.BlockSpec(memory_space=pl.ANY),
    ],
    out_shape=(
        jax.ShapeDtypeStruct((1, 1), jnp.float32),
        jax.ShapeDtypeStruct((1, 1), jnp.float32),
        jax.ShapeDtypeStruct((1, 1), jnp.float32),
    ),
    scratch_shapes=[
        pltpu.VMEM((TC_ROWS, TC_ROW), jnp.float32),
        pltpu.VMEM((TC_ROWS, TC_ROW), jnp.float32),
        pltpu.SemaphoreType.DMA,
        pltpu.SemaphoreType.DMA,
    ],
)


def _tc2_body(part_ref, nll_ref, ap_ref, l1tc_ref, loss_ref, nll_out_ref):
    nll = nll_ref[0, 0]
    l1 = jnp.sum(part_ref[...]) + l1tc_ref[0, 0]
    loss = nll + (ALPHA / 2.0) * l1 + ap_ref[0, 0]
    loss_ref[...] = loss.reshape(1, 1)
    nll_out_ref[...] = nll.reshape(1, 1)


_tc2_call = pl.pallas_call(
    _tc2_body,
    out_shape=(
        jax.ShapeDtypeStruct((1, 1), jnp.float32),
        jax.ShapeDtypeStruct((1, 1), jnp.float32),
    ),
)


def kernel(logits, labels, params, lengths, attn_weights):
    del lengths
    partials = _l1_partials_kernel()(params)
    nll, ap, l1tc = _tc1_call(
        logits, labels.reshape(B, 1), attn_weights.reshape(B, L), params)
    loss, nll_out = _tc2_call(partials, nll, ap, l1tc)
    return (loss.reshape(()), nll_out.reshape(()))
